# Initial kernel scaffold; baseline (speedup 1.0000x reference)
import jax
import jax.numpy as jnp
from jax.experimental import pallas as pl


def kernel(original_x, original_edge_index, perm):
    raise NotImplementedError("write your pallas kernel here")


# speedup vs baseline: 6.4049x; 6.4049x over previous
import functools

import jax
import jax.numpy as jnp
from jax import lax
from jax.experimental import pallas as pl
from jax.experimental.pallas import tpu as pltpu
from jax.experimental.pallas import tpu_sc as plsc

N_NODES = 10000
N_EDGES = 320000
D_FEAT = 128
N_PERM = 5000

NC = 2
NS = 16
NW = NC * NS

CHUNK = 128
EDGE_CHUNKS = 80
E_PAD = NW * EDGE_CHUNKS * CHUNK
ACC_ROWS = 10240
TRASH = N_NODES
ROWS_PER_TILE = ACC_ROWS // NS
PERM_CHUNKS = 2
P_PAD = NW * PERM_CHUNKS * CHUNK

_mesh = functools.partial(
    plsc.VectorSubcoreMesh, core_axis_name="c", subcore_axis_name="s")


@functools.partial(
    pl.kernel,
    mesh=_mesh(),
    out_type=jax.ShapeDtypeStruct((NC * ACC_ROWS, D_FEAT), jnp.float32),
    scratch_types=[
        pltpu.VMEM((EDGE_CHUNKS, CHUNK), jnp.int32),
        pltpu.VMEM((EDGE_CHUNKS, CHUNK), jnp.int32),
        pltpu.VMEM((EDGE_CHUNKS, CHUNK), jnp.int32),
        pltpu.VMEM((CHUNK, D_FEAT), jnp.float32),
        pltpu.VMEM_SHARED((ACC_ROWS, D_FEAT), jnp.float32),
        pltpu.SemaphoreType.DMA,
    ],
)
def _edge_accumulate(row_hbm, col_hbm, x_hbm, out_hbm,
                     row_v, col_v, tgt_v, xg, acc, sem):
    c = lax.axis_index("c")
    s = lax.axis_index("s")
    wid = c * NS + s

    def _zbody(r, _):
        for j in range(D_FEAT // 16):
            xg[r, pl.ds(j * 16, 16)] = jnp.zeros((16,), jnp.float32)
        return 0
    lax.fori_loop(0, CHUNK, _zbody, 0)
    for b in range(ROWS_PER_TILE // CHUNK):
        pltpu.sync_copy(xg, acc.at[pl.ds(s * ROWS_PER_TILE + b * CHUNK, CHUNK)])
    plsc.subcore_barrier()

    pltpu.sync_copy(row_hbm.at[wid], row_v)
    pltpu.sync_copy(col_hbm.at[wid], col_v)

    def _tbody(k, _):
        for j in range(CHUNK // 16):
            r = row_v[k, pl.ds(j * 16, 16)]
            cc = col_v[k, pl.ds(j * 16, 16)]
            tgt_v[k, pl.ds(j * 16, 16)] = jnp.where(
                r == cc, jnp.full((16,), TRASH, jnp.int32), r)
        return 0
    lax.fori_loop(0, EDGE_CHUNKS, _tbody, 0)

    def _ebody(k, _):
        pltpu.async_copy(x_hbm.at[col_v.at[k]], xg, sem).wait()
        pltpu.sync_copy(xg, acc.at[tgt_v.at[k]], add=True)
        return 0
    lax.fori_loop(0, EDGE_CHUNKS, _ebody, 0)
    plsc.subcore_barrier()

    for b in range(ROWS_PER_TILE // CHUNK):
        base = s * ROWS_PER_TILE + b * CHUNK
        pltpu.sync_copy(acc.at[pl.ds(base, CHUNK)], xg)
        pltpu.sync_copy(xg, out_hbm.at[pl.ds(c * ACC_ROWS + base, CHUNK)])


@functools.partial(
    pl.kernel,
    mesh=_mesh(),
    out_type=jax.ShapeDtypeStruct((P_PAD, D_FEAT), jnp.float32),
    scratch_types=[
        pltpu.VMEM((PERM_CHUNKS, CHUNK), jnp.int32),
        pltpu.VMEM((PERM_CHUNKS, CHUNK), jnp.int32),
        pltpu.VMEM((CHUNK, D_FEAT), jnp.float32),
        pltpu.VMEM((CHUNK, D_FEAT), jnp.float32),
        pltpu.VMEM((CHUNK, D_FEAT), jnp.float32),
        pltpu.SemaphoreType.DMA,
    ],
)
def _combine(parts_hbm, x_hbm, perm_hbm, out_hbm,
             pi, pi2, ga, gb, gx, sem):
    c = lax.axis_index("c")
    s = lax.axis_index("s")
    wid = c * NS + s

    pltpu.sync_copy(perm_hbm.at[wid], pi)
    for k in range(PERM_CHUNKS):
        for j in range(CHUNK // 16):
            pi2[k, pl.ds(j * 16, 16)] = (
                pi[k, pl.ds(j * 16, 16)] + jnp.full((16,), ACC_ROWS, jnp.int32))

    for k in range(PERM_CHUNKS):
        pltpu.async_copy(parts_hbm.at[pi.at[k]], ga, sem).wait()
        pltpu.async_copy(parts_hbm.at[pi2.at[k]], gb, sem).wait()
        pltpu.async_copy(x_hbm.at[pi.at[k]], gx, sem).wait()

        def _sbody(r, _):
            for j in range(D_FEAT // 16):
                sl = pl.ds(j * 16, 16)
                ga[r, sl] = ga[r, sl] + gb[r, sl] + gx[r, sl]
            return 0
        lax.fori_loop(0, CHUNK, _sbody, 0)
        pltpu.sync_copy(
            ga, out_hbm.at[pl.ds((wid * PERM_CHUNKS + k) * CHUNK, CHUNK)])


def kernel(original_x, original_edge_index, perm):
    row = original_edge_index[0]
    col = original_edge_index[1]
    zpad = jnp.zeros((E_PAD - N_EDGES,), jnp.int32)
    row3 = jnp.concatenate([row, zpad]).reshape(NW, EDGE_CHUNKS, CHUNK)
    col3 = jnp.concatenate([col, zpad]).reshape(NW, EDGE_CHUNKS, CHUNK)
    perm3 = jnp.concatenate(
        [perm, jnp.zeros((P_PAD - N_PERM,), perm.dtype)]
    ).reshape(NW, PERM_CHUNKS, CHUNK)

    parts = _edge_accumulate(row3, col3, original_x)
    out = _combine(parts, original_x, perm3)
    return out[:N_PERM]


# speedup vs baseline: 6.9650x; 1.0875x over previous
import functools

import jax
import jax.numpy as jnp
from jax import lax
from jax.experimental import pallas as pl
from jax.experimental.pallas import tpu as pltpu
from jax.experimental.pallas import tpu_sc as plsc

N_NODES = 10000
N_EDGES = 320000
D_FEAT = 128
N_PERM = 5000

NC = 2
NS = 16
NW = NC * NS

CHUNK = 128
EDGE_CHUNKS = 80
E_PAD = NW * EDGE_CHUNKS * CHUNK
ACC_ROWS = 10240
TRASH = N_NODES
ROWS_PER_TILE = ACC_ROWS // NS
PERM_CHUNKS = 2
P_PAD = NW * PERM_CHUNKS * CHUNK

_mesh = functools.partial(
    plsc.VectorSubcoreMesh, core_axis_name="c", subcore_axis_name="s")


@functools.partial(
    pl.kernel,
    mesh=_mesh(),
    out_type=jax.ShapeDtypeStruct((NC * ACC_ROWS, D_FEAT), jnp.float32),
    scratch_types=[
        pltpu.VMEM((EDGE_CHUNKS, CHUNK), jnp.int32),
        pltpu.VMEM((2, CHUNK), jnp.int32),
        pltpu.VMEM((2, CHUNK), jnp.int32),
        pltpu.VMEM((CHUNK, D_FEAT), jnp.float32),
        pltpu.VMEM((CHUNK, D_FEAT), jnp.float32),
        pltpu.VMEM_SHARED((ACC_ROWS, D_FEAT), jnp.float32),
        pltpu.SemaphoreType.DMA,
        pltpu.SemaphoreType.DMA,
        pltpu.SemaphoreType.DMA,
        pltpu.SemaphoreType.DMA,
    ],
)
def _edge_accumulate(row_hbm, col_hbm, x_hbm, out_hbm,
                     col_v, row_v, tgt_v, xg, xg1, acc,
                     gs0, gs1, rs0, rs1):
    c = lax.axis_index("c")
    s = lax.axis_index("s")
    wid = c * NS + s

    def _zbody(r, _):
        for j in range(D_FEAT // 16):
            xg[r, pl.ds(j * 16, 16)] = jnp.zeros((16,), jnp.float32)
        return 0
    lax.fori_loop(0, CHUNK, _zbody, 0)
    for b in range(ROWS_PER_TILE // CHUNK):
        pltpu.sync_copy(xg, acc.at[pl.ds(s * ROWS_PER_TILE + b * CHUNK, CHUNK)])
    plsc.subcore_barrier()

    pltpu.sync_copy(col_hbm.at[wid], col_v)

    bufs = (xg, xg1)
    gsems = (gs0, gs1)
    rsems = (rs0, rs1)
    n_groups = EDGE_CHUNKS // 2
    rbase = wid * EDGE_CHUNKS
    for b in range(2):
        pltpu.async_copy(x_hbm.at[col_v.at[b]], bufs[b], gsems[b])
        pltpu.async_copy(row_hbm.at[rbase + b], row_v.at[b], rsems[b])

    def _ebody(g, _):
        for b in range(2):
            k = g * 2 + b
            pltpu.make_async_copy(
                row_hbm.at[rbase + k], row_v.at[b], rsems[b]).wait()
            for j in range(CHUNK // 16):
                sl = pl.ds(j * 16, 16)
                r = row_v[b, sl]
                cc = col_v[k, sl]
                tgt_v[b, sl] = jnp.where(
                    r == cc, jnp.full((16,), TRASH, jnp.int32), r)

            @pl.when(g < n_groups - 1)
            def _():
                pltpu.async_copy(
                    row_hbm.at[rbase + k + 2], row_v.at[b], rsems[b])
            pltpu.make_async_copy(
                x_hbm.at[col_v.at[k]], bufs[b], gsems[b]).wait()
            pltpu.sync_copy(bufs[b], acc.at[tgt_v.at[b]], add=True)

            @pl.when(g < n_groups - 1)
            def _():
                pltpu.async_copy(x_hbm.at[col_v.at[k + 2]], bufs[b], gsems[b])
        return 0
    lax.fori_loop(0, n_groups, _ebody, 0)
    plsc.subcore_barrier()

    for b in range(ROWS_PER_TILE // CHUNK):
        base = s * ROWS_PER_TILE + b * CHUNK
        pltpu.sync_copy(acc.at[pl.ds(base, CHUNK)], xg)
        pltpu.sync_copy(xg, out_hbm.at[pl.ds(c * ACC_ROWS + base, CHUNK)])


@functools.partial(
    pl.kernel,
    mesh=_mesh(),
    out_type=jax.ShapeDtypeStruct((P_PAD, D_FEAT), jnp.float32),
    scratch_types=[
        pltpu.VMEM((PERM_CHUNKS, CHUNK), jnp.int32),
        pltpu.VMEM((PERM_CHUNKS, CHUNK), jnp.int32),
        pltpu.VMEM((CHUNK, D_FEAT), jnp.float32),
        pltpu.VMEM((CHUNK, D_FEAT), jnp.float32),
        pltpu.VMEM((CHUNK, D_FEAT), jnp.float32),
        pltpu.SemaphoreType.DMA,
    ],
)
def _combine(parts_hbm, x_hbm, perm_hbm, out_hbm,
             pi, pi2, ga, gb, gx, sem):
    c = lax.axis_index("c")
    s = lax.axis_index("s")
    wid = c * NS + s

    pltpu.sync_copy(perm_hbm.at[wid], pi)
    for k in range(PERM_CHUNKS):
        for j in range(CHUNK // 16):
            pi2[k, pl.ds(j * 16, 16)] = (
                pi[k, pl.ds(j * 16, 16)] + jnp.full((16,), ACC_ROWS, jnp.int32))

    for k in range(PERM_CHUNKS):
        pltpu.async_copy(parts_hbm.at[pi.at[k]], ga, sem).wait()
        pltpu.async_copy(parts_hbm.at[pi2.at[k]], gb, sem).wait()
        pltpu.async_copy(x_hbm.at[pi.at[k]], gx, sem).wait()

        def _sbody(r, _):
            for j in range(D_FEAT // 16):
                sl = pl.ds(j * 16, 16)
                ga[r, sl] = ga[r, sl] + gb[r, sl] + gx[r, sl]
            return 0
        lax.fori_loop(0, CHUNK, _sbody, 0)
        pltpu.sync_copy(
            ga, out_hbm.at[pl.ds((wid * PERM_CHUNKS + k) * CHUNK, CHUNK)])


def kernel(original_x, original_edge_index, perm):
    row = original_edge_index[0]
    col = original_edge_index[1]
    zpad = jnp.zeros((E_PAD - N_EDGES,), jnp.int32)
    row3 = jnp.concatenate([row, zpad]).reshape(NW * EDGE_CHUNKS, CHUNK)
    col3 = jnp.concatenate([col, zpad]).reshape(NW, EDGE_CHUNKS, CHUNK)
    perm3 = jnp.concatenate(
        [perm, jnp.zeros((P_PAD - N_PERM,), perm.dtype)]
    ).reshape(NW, PERM_CHUNKS, CHUNK)

    parts = _edge_accumulate(row3, col3, original_x)
    out = _combine(parts, original_x, perm3)
    return out[:N_PERM]


# speedup vs baseline: 7.7624x; 1.1145x over previous
import functools

import jax
import jax.numpy as jnp
from jax import lax
from jax.experimental import pallas as pl
from jax.experimental.pallas import tpu as pltpu
from jax.experimental.pallas import tpu_sc as plsc

N_NODES = 10000
N_EDGES = 320000
D_FEAT = 128
N_PERM = 5000

NC = 2
NS = 16
NW = NC * NS

CHUNK = 128
EDGE_CHUNKS = 80
E_PAD = NW * EDGE_CHUNKS * CHUNK
ACC_ROWS = 10240
TRASH = N_NODES
ROWS_PER_TILE = ACC_ROWS // NS
PERM_CHUNKS = 2
P_PAD = NW * PERM_CHUNKS * CHUNK

_mesh = functools.partial(
    plsc.VectorSubcoreMesh, core_axis_name="c", subcore_axis_name="s")


@functools.partial(
    pl.kernel,
    mesh=_mesh(),
    out_type=jax.ShapeDtypeStruct((NC * ACC_ROWS, D_FEAT), jnp.float32),
    scratch_types=[
        pltpu.VMEM((EDGE_CHUNKS, CHUNK), jnp.int32),
        pltpu.VMEM((2, CHUNK), jnp.int32),
        pltpu.VMEM((2, CHUNK), jnp.int32),
        pltpu.VMEM((CHUNK, D_FEAT), jnp.float32),
        pltpu.VMEM((CHUNK, D_FEAT), jnp.float32),
        pltpu.VMEM_SHARED((ACC_ROWS, D_FEAT), jnp.float32),
        pltpu.SemaphoreType.DMA,
        pltpu.SemaphoreType.DMA,
        pltpu.SemaphoreType.DMA,
        pltpu.SemaphoreType.DMA,
        pltpu.SemaphoreType.DMA,
        pltpu.SemaphoreType.DMA,
    ],
)
def _edge_accumulate(row_hbm, col_hbm, x_hbm, out_hbm,
                     col_v, row_v, tgt_v, xg, xg1, acc,
                     gs0, gs1, rs0, rs1, ss0, ss1):
    c = lax.axis_index("c")
    s = lax.axis_index("s")
    wid = c * NS + s

    def _zbody(r, _):
        for j in range(D_FEAT // 16):
            xg[r, pl.ds(j * 16, 16)] = jnp.zeros((16,), jnp.float32)
        return 0
    lax.fori_loop(0, CHUNK, _zbody, 0)
    for b in range(ROWS_PER_TILE // CHUNK):
        pltpu.sync_copy(xg, acc.at[pl.ds(s * ROWS_PER_TILE + b * CHUNK, CHUNK)])
    plsc.subcore_barrier()

    pltpu.sync_copy(col_hbm.at[wid], col_v)

    bufs = (xg, xg1)
    gsems = (gs0, gs1)
    rsems = (rs0, rs1)
    ssems = (ss0, ss1)
    n_groups = EDGE_CHUNKS // 2
    rbase = wid * EDGE_CHUNKS
    for b in range(2):
        pltpu.async_copy(x_hbm.at[col_v.at[b]], bufs[b], gsems[b])
        pltpu.async_copy(row_hbm.at[rbase + b], row_v.at[b], rsems[b])

    def _ebody(g, _):
        for b in range(2):
            k = g * 2 + b

            @pl.when(g >= 1)
            def _():
                pltpu.make_async_copy(
                    bufs[b], acc.at[tgt_v.at[b]], ssems[b]).wait()
                pltpu.async_copy(x_hbm.at[col_v.at[k]], bufs[b], gsems[b])

            pltpu.make_async_copy(
                row_hbm.at[rbase + k], row_v.at[b], rsems[b]).wait()
            for j in range(CHUNK // 16):
                sl = pl.ds(j * 16, 16)
                r = row_v[b, sl]
                cc = col_v[k, sl]
                tgt_v[b, sl] = jnp.where(
                    r == cc, jnp.full((16,), TRASH, jnp.int32), r)

            @pl.when(g < n_groups - 1)
            def _():
                pltpu.async_copy(
                    row_hbm.at[rbase + k + 2], row_v.at[b], rsems[b])

            pltpu.make_async_copy(
                x_hbm.at[col_v.at[k]], bufs[b], gsems[b]).wait()
            pltpu.async_copy(
                bufs[b], acc.at[tgt_v.at[b]], ssems[b], add=True)
        return 0
    lax.fori_loop(0, n_groups, _ebody, 0)
    for b in range(2):
        pltpu.make_async_copy(bufs[b], acc.at[tgt_v.at[b]], ssems[b]).wait()
    plsc.subcore_barrier()

    for b in range(ROWS_PER_TILE // CHUNK):
        base = s * ROWS_PER_TILE + b * CHUNK
        pltpu.sync_copy(acc.at[pl.ds(base, CHUNK)], xg)
        pltpu.sync_copy(xg, out_hbm.at[pl.ds(c * ACC_ROWS + base, CHUNK)])


@functools.partial(
    pl.kernel,
    mesh=_mesh(),
    out_type=jax.ShapeDtypeStruct((P_PAD, D_FEAT), jnp.float32),
    scratch_types=[
        pltpu.VMEM((PERM_CHUNKS, CHUNK), jnp.int32),
        pltpu.VMEM((PERM_CHUNKS, CHUNK), jnp.int32),
        pltpu.VMEM((CHUNK, D_FEAT), jnp.float32),
        pltpu.VMEM((CHUNK, D_FEAT), jnp.float32),
        pltpu.VMEM((CHUNK, D_FEAT), jnp.float32),
        pltpu.SemaphoreType.DMA,
        pltpu.SemaphoreType.DMA,
        pltpu.SemaphoreType.DMA,
    ],
)
def _combine(parts_hbm, x_hbm, perm_hbm, out_hbm,
             pi, pi2, ga, gb, gx, sa, sb, sx):
    c = lax.axis_index("c")
    s = lax.axis_index("s")
    wid = c * NS + s

    pltpu.sync_copy(perm_hbm.at[wid], pi)
    for k in range(PERM_CHUNKS):
        for j in range(CHUNK // 16):
            pi2[k, pl.ds(j * 16, 16)] = (
                pi[k, pl.ds(j * 16, 16)] + jnp.full((16,), ACC_ROWS, jnp.int32))

    for k in range(PERM_CHUNKS):
        ca = pltpu.async_copy(parts_hbm.at[pi.at[k]], ga, sa)
        cb = pltpu.async_copy(parts_hbm.at[pi2.at[k]], gb, sb)
        cx = pltpu.async_copy(x_hbm.at[pi.at[k]], gx, sx)
        ca.wait()
        cb.wait()
        cx.wait()

        def _sbody(r, _):
            for j in range(D_FEAT // 16):
                sl = pl.ds(j * 16, 16)
                ga[r, sl] = ga[r, sl] + gb[r, sl] + gx[r, sl]
            return 0
        lax.fori_loop(0, CHUNK, _sbody, 0)
        pltpu.sync_copy(
            ga, out_hbm.at[pl.ds((wid * PERM_CHUNKS + k) * CHUNK, CHUNK)])


def kernel(original_x, original_edge_index, perm):
    row = original_edge_index[0]
    col = original_edge_index[1]
    zpad = jnp.zeros((E_PAD - N_EDGES,), jnp.int32)
    row3 = jnp.concatenate([row, zpad]).reshape(NW * EDGE_CHUNKS, CHUNK)
    col3 = jnp.concatenate([col, zpad]).reshape(NW, EDGE_CHUNKS, CHUNK)
    perm3 = jnp.concatenate(
        [perm, jnp.zeros((P_PAD - N_PERM,), perm.dtype)]
    ).reshape(NW, PERM_CHUNKS, CHUNK)

    parts = _edge_accumulate(row3, col3, original_x)
    out = _combine(parts, original_x, perm3)
    return out[:N_PERM]
